# baseline (speedup 1.0000x reference)
import jax
import jax.numpy as jnp
from jax.experimental import pallas as pl
from jax.experimental.pallas import tpu as pltpu

_NC = 8


def _body(pos_ref, prev_hbm, cur_ref, out_hbm, q_ref, copy_sems, row_sem):
    BH = prev_hbm.shape[0]
    ch = BH // _NC
    for c in range(_NC):
        pltpu.make_async_copy(
            prev_hbm.at[pl.ds(c * ch, ch)],
            out_hbm.at[pl.ds(c * ch, ch)],
            copy_sems.at[c],
        ).start()
    q_ref[...] = cur_ref[...].astype(jnp.float8_e4m3fn).astype(jnp.float32)
    for c in range(_NC):
        pltpu.make_async_copy(
            prev_hbm.at[pl.ds(c * ch, ch)],
            out_hbm.at[pl.ds(c * ch, ch)],
            copy_sems.at[c],
        ).wait()
    pos = pos_ref[0]
    row = pltpu.make_async_copy(
        q_ref, out_hbm.at[:, pl.ds(pos, 1), :], row_sem
    )
    row.start()
    row.wait()


def kernel(prev, cur, dim, idx):
    B, H, S, D = prev.shape
    BH = B * H
    prev3 = prev.reshape(BH, S, D)
    cur3 = cur.reshape(BH, 1, D)
    pos = (idx[0] - dim + 1).astype(jnp.int32).reshape((1,))
    out = pl.pallas_call(
        _body,
        in_specs=[
            pl.BlockSpec(memory_space=pltpu.SMEM),
            pl.BlockSpec(memory_space=pl.ANY),
            pl.BlockSpec(memory_space=pltpu.VMEM),
        ],
        out_specs=pl.BlockSpec(memory_space=pl.ANY),
        out_shape=jax.ShapeDtypeStruct((BH, S, D), prev.dtype),
        scratch_shapes=[
            pltpu.VMEM((BH, 1, D), jnp.float32),
            pltpu.SemaphoreType.DMA((_NC,)),
            pltpu.SemaphoreType.DMA,
        ],
    )(pos, prev3, cur3)
    return out.reshape(B, H, S, D)


# speedup vs baseline: 48.1918x; 48.1918x over previous
import jax
import jax.numpy as jnp
from jax.experimental import pallas as pl
from jax.experimental.pallas import tpu as pltpu


def _body(pos_ref, prev_ref, cur_ref, out_ref):
    pos = pos_ref[0]
    x = prev_ref[...]
    q = cur_ref[...].astype(jnp.float8_e4m3fn).astype(x.dtype)
    row = jax.lax.broadcasted_iota(jnp.int32, x.shape, 1)
    out_ref[...] = jnp.where(row == pos, q, x)


def kernel(prev, cur, dim, idx):
    B, H, S, D = prev.shape
    BH = B * H
    BLK = 8
    prev3 = prev.reshape(BH, S, D)
    cur3 = cur.reshape(BH, 1, D)
    pos = (idx[0] - dim + 1).astype(jnp.int32).reshape((1,))
    grid_spec = pltpu.PrefetchScalarGridSpec(
        num_scalar_prefetch=1,
        grid=(BH // BLK,),
        in_specs=[
            pl.BlockSpec((BLK, S, D), lambda i, pos_ref: (i, 0, 0)),
            pl.BlockSpec((BLK, 1, D), lambda i, pos_ref: (i, 0, 0)),
        ],
        out_specs=pl.BlockSpec((BLK, S, D), lambda i, pos_ref: (i, 0, 0)),
    )
    out = pl.pallas_call(
        _body,
        grid_spec=grid_spec,
        out_shape=jax.ShapeDtypeStruct((BH, S, D), prev.dtype),
        compiler_params=pltpu.CompilerParams(
            dimension_semantics=("parallel",),
        ),
    )(pos, prev3, cur3)
    return out.reshape(B, H, S, D)


# speedup vs baseline: 48.3443x; 1.0032x over previous
import jax
import jax.numpy as jnp
from jax.experimental import pallas as pl
from jax.experimental.pallas import tpu as pltpu


def _body(pos_ref, prev_ref, cur_ref, out_ref):
    out_ref[...] = prev_ref[...]
    pos = pos_ref[0]
    q = cur_ref[...].astype(jnp.float8_e4m3fn).astype(out_ref.dtype)
    out_ref[:, pl.ds(pos, 1), :] = q


def kernel(prev, cur, dim, idx):
    B, H, S, D = prev.shape
    BH = B * H
    BLK = 8
    prev3 = prev.reshape(BH, S, D)
    cur3 = cur.reshape(BH, 1, D)
    pos = (idx[0] - dim + 1).astype(jnp.int32).reshape((1,))
    grid_spec = pltpu.PrefetchScalarGridSpec(
        num_scalar_prefetch=1,
        grid=(BH // BLK,),
        in_specs=[
            pl.BlockSpec((BLK, S, D), lambda i, pos_ref: (i, 0, 0)),
            pl.BlockSpec((BLK, 1, D), lambda i, pos_ref: (i, 0, 0)),
        ],
        out_specs=pl.BlockSpec((BLK, S, D), lambda i, pos_ref: (i, 0, 0)),
    )
    out = pl.pallas_call(
        _body,
        grid_spec=grid_spec,
        out_shape=jax.ShapeDtypeStruct((BH, S, D), prev.dtype),
        compiler_params=pltpu.CompilerParams(
            dimension_semantics=("parallel",),
        ),
    )(pos, prev3, cur3)
    return out.reshape(B, H, S, D)
